# baseline (speedup 1.0000x reference)
import functools

import jax
import jax.numpy as jnp
from jax import lax
from jax.experimental import pallas as pl
from jax.experimental.pallas import tpu as pltpu
from jax.experimental.pallas import tpu_sc as plsc

K = 8192
D = 256
L = 512
B = 16
H = 128
N = B * L
CC = 0.25

RT = 512
KT = 2048
RT2 = 512
KT2 = 1024
CCHUNK = 16384


def _bf(v):
    return v.astype(jnp.bfloat16)


def _main_body(inp_ref, w1_ref, b1_ref, w2_ref, b2_ref, e_ref, et_ref,
               x_ref, idx_ref, se_ref, etb_ref):
    i = pl.program_id(0)

    @pl.when(i == 0)
    def _():
        ee = e_ref[...]
        se_ref[...] = jnp.sum(ee * ee, axis=1).reshape(1, K)
        etb_ref[...] = _bf(et_ref[...])

    inp = inp_ref[...]
    h = jnp.maximum(inp * w1_ref[...] + b1_ref[...], 0.0)
    x = lax.dot_general(_bf(h), _bf(w2_ref[...]),
                        (((1,), (0,)), ((), ())),
                        preferred_element_type=jnp.float32) + b2_ref[...]
    x_ref[...] = x
    s = jnp.sum(x * x, axis=1, keepdims=True)
    xb = _bf(x)

    best_v = None
    best_i = None
    for c in range(K // KT):
        ec = etb_ref[:, pl.ds(c * KT, KT)]
        mm = lax.dot_general(xb, ec, (((1,), (0,)), ((), ())),
                             preferred_element_type=jnp.float32)
        d = (s + se_ref[0:1, pl.ds(c * KT, KT)]) - 2.0 * mm
        m = jnp.min(d, axis=1, keepdims=True)
        io = lax.broadcasted_iota(jnp.int32, (RT, KT), 1) + c * KT
        ci = jnp.min(jnp.where(d == m, io, K), axis=1, keepdims=True)
        if c == 0:
            best_v, best_i = m, ci
        else:
            upd = m < best_v
            best_i = jnp.where(upd, ci, best_i)
            best_v = jnp.where(upd, m, best_v)
    idx_ref[...] = best_i


def _run_main(inp2d, W1, b1, W2, b2, E, ET):
    return pl.pallas_call(
        _main_body,
        grid=(N // RT,),
        in_specs=[
            pl.BlockSpec((RT, 1), lambda i: (i, 0)),
            pl.BlockSpec((1, H), lambda i: (0, 0)),
            pl.BlockSpec((1, H), lambda i: (0, 0)),
            pl.BlockSpec((H, D), lambda i: (0, 0)),
            pl.BlockSpec((1, D), lambda i: (0, 0)),
            pl.BlockSpec((K, D), lambda i: (0, 0)),
            pl.BlockSpec((D, K), lambda i: (0, 0)),
        ],
        out_specs=[
            pl.BlockSpec((RT, D), lambda i: (i, 0)),
            pl.BlockSpec((RT, 1), lambda i: (i, 0)),
        ],
        out_shape=[
            jax.ShapeDtypeStruct((N, D), jnp.float32),
            jax.ShapeDtypeStruct((N, 1), jnp.int32),
        ],
        scratch_shapes=[
            pltpu.VMEM((1, K), jnp.float32),
            pltpu.VMEM((D, K), jnp.bfloat16),
        ],
    )(inp2d, W1, b1, W2, b2, E, ET)


def _onehot_body(idx_ref, oh_ref, cnt_ref):
    j = pl.program_id(0)
    i = pl.program_id(1)
    ids = idx_ref[...]
    io = lax.broadcasted_iota(jnp.int32, (RT2, KT2), 1) + j * KT2
    oh = (ids == io).astype(jnp.float32)
    oh_ref[...] = oh

    @pl.when(i == 0)
    def _():
        cnt_ref[...] = jnp.zeros((1, KT2), jnp.float32)

    cnt_ref[...] += jnp.sum(oh, axis=0, keepdims=True)


def _run_onehot(idx):
    return pl.pallas_call(
        _onehot_body,
        grid=(K // KT2, N // RT2),
        in_specs=[pl.BlockSpec((RT2, 1), lambda j, i: (i, 0))],
        out_specs=[
            pl.BlockSpec((RT2, KT2), lambda j, i: (i, j)),
            pl.BlockSpec((1, KT2), lambda j, i: (0, j)),
        ],
        out_shape=[
            jax.ShapeDtypeStruct((N, K), jnp.float32),
            jax.ShapeDtypeStruct((1, K), jnp.float32),
        ],
    )(idx)


def _make_sc_gather():
    info = plsc.get_sparse_core_info()
    nw = info.num_cores * info.num_subcores
    rows_per_w = N // nw
    mesh = plsc.VectorSubcoreMesh(core_axis_name="c", subcore_axis_name="s")

    @functools.partial(
        pl.kernel, mesh=mesh,
        out_type=jax.ShapeDtypeStruct((N, D), jnp.float32),
        scratch_types=[
            pltpu.VMEM((rows_per_w,), jnp.int32),
            pltpu.VMEM((rows_per_w, D), jnp.float32),
            pltpu.SemaphoreType.DMA,
        ],
    )
    def gather_k(e_hbm, idx_hbm, out_hbm, idx_v, rows_v, sem):
        wid = lax.axis_index("s") * info.num_cores + lax.axis_index("c")
        base = wid * rows_per_w
        pltpu.sync_copy(idx_hbm.at[pl.ds(base, rows_per_w)], idx_v)
        pltpu.async_copy(e_hbm.at[idx_v], rows_v, sem).wait()
        pltpu.sync_copy(rows_v, out_hbm.at[pl.ds(base, rows_per_w)])

    return gather_k


def _qst_body(x_ref, q_ref, qst_ref, loss_ref):
    i = pl.program_id(0)
    x = x_ref[...]
    q = q_ref[...]
    qst_ref[...] = x + (q - x)
    diff = q - x

    @pl.when(i == 0)
    def _():
        loss_ref[...] = jnp.zeros((1, 1), jnp.float32)

    loss_ref[...] += jnp.sum(diff * diff).reshape(1, 1)


def _run_qst(x, q):
    return pl.pallas_call(
        _qst_body,
        grid=(N // RT,),
        in_specs=[
            pl.BlockSpec((RT, D), lambda i: (i, 0)),
            pl.BlockSpec((RT, D), lambda i: (i, 0)),
        ],
        out_specs=[
            pl.BlockSpec((RT, D), lambda i: (i, 0)),
            pl.BlockSpec((1, 1), lambda i: (0, 0)),
        ],
        out_shape=[
            jax.ShapeDtypeStruct((N, D), jnp.float32),
            jax.ShapeDtypeStruct((1, 1), jnp.float32),
        ],
    )(x, q)


def _cls_body(q_ref, wc_ref, bc_ref, cls_ref, acc_ref):
    c = pl.program_id(0)

    @pl.when(c == 0)
    def _():
        acc_ref[...] = jnp.zeros((B, 16), jnp.float32)

    z = lax.dot_general(_bf(q_ref[...]), _bf(wc_ref[...]),
                        (((1,), (0,)), ((), ())),
                        preferred_element_type=jnp.float32)
    acc_ref[:, :10] += z

    @pl.when(c == pl.num_programs(0) - 1)
    def _():
        zz = acc_ref[:, :10] + bc_ref[...]
        cls_ref[...] = 1.0 / (1.0 + jnp.exp(-zz))


def _run_cls(qflat, Wc, bc):
    return pl.pallas_call(
        _cls_body,
        grid=(L * D // CCHUNK,),
        in_specs=[
            pl.BlockSpec((B, CCHUNK), lambda c: (0, c)),
            pl.BlockSpec((CCHUNK, 10), lambda c: (c, 0)),
            pl.BlockSpec((1, 10), lambda c: (0, 0)),
        ],
        out_specs=pl.BlockSpec((B, 10), lambda c: (0, 0)),
        out_shape=jax.ShapeDtypeStruct((B, 10), jnp.float32),
        scratch_shapes=[pltpu.VMEM((B, 16), jnp.float32)],
    )(qflat, Wc, bc)


def _scalars_body(cnt_ref, loss_ref, vq_ref, perp_ref):
    p = cnt_ref[...] * (1.0 / K)
    t = p * jnp.log(p + 1e-10)
    perp_ref[...] = jnp.exp(-jnp.sum(t)).reshape(1, 1)
    m = loss_ref[0, 0] * (1.0 / (B * L * D))
    vq_ref[...] = (m + CC * m).reshape(1, 1)


def _run_scalars(counts, loss_sum):
    return pl.pallas_call(
        _scalars_body,
        out_shape=[
            jax.ShapeDtypeStruct((1, 1), jnp.float32),
            jax.ShapeDtypeStruct((1, 1), jnp.float32),
        ],
    )(counts, loss_sum)


def kernel(inputs, E, W1, b1, W2, b2, Wc, bc):
    inp2d = inputs.reshape(N, 1)
    ET = E.T
    x, idx = _run_main(inp2d, W1, b1.reshape(1, H), W2, b2.reshape(1, D),
                       E, ET)
    encodings, counts = _run_onehot(idx)
    q = _make_sc_gather()(E, idx.reshape(N))
    qst, loss_sum = _run_qst(x, q)
    cls = _run_cls(q.reshape(B, L * D), Wc, bc.reshape(1, 10))
    vq, perp = _run_scalars(counts, loss_sum)
    return (vq.reshape(()), qst.reshape(B, L * D), perp.reshape(()),
            encodings, cls)
